# initial kernel scaffold (unmeasured)
import jax
import jax.numpy as jnp
from jax import lax
from jax.experimental import pallas as pl
from jax.experimental.pallas import tpu as pltpu

M = 2048
D = 2048
HALF = M // 2


def kernel(partial, resid, gamma):
    gamma2d = gamma.reshape(1, D)

    def body(p_ref, r_ref, g_ref, out_ref, xrecv_ref, sems):
        my_x = lax.axis_index("x")
        my_y = lax.axis_index("y")
        x_nbr = (1 - my_x, my_y)
        y_nbr = (my_x, 1 - my_y)

        barrier = pltpu.get_barrier_semaphore()
        for nbr in (x_nbr, y_nbr):
            pl.semaphore_signal(
                barrier, inc=1, device_id=nbr,
                device_id_type=pl.DeviceIdType.MESH,
            )
        pl.semaphore_wait(barrier, 2)

        row0 = my_y * HALF

        x_rdma = pltpu.make_async_remote_copy(
            src_ref=p_ref.at[0, pl.ds(row0, HALF), :],
            dst_ref=xrecv_ref,
            send_sem=sems.at[0],
            recv_sem=sems.at[1],
            device_id=x_nbr,
            device_id_type=pl.DeviceIdType.MESH,
        )
        x_rdma.start()
        x_rdma.wait()

        y = (
            p_ref[0, pl.ds(row0, HALF), :]
            + xrecv_ref[...]
            + r_ref[pl.ds(row0, HALF), :]
        )
        ms = jnp.mean(y * y, axis=-1, keepdims=True)
        out_ref[pl.ds(row0, HALF), :] = (y * lax.rsqrt(ms + 1e-6)) * g_ref[...]

        y_rdma = pltpu.make_async_remote_copy(
            src_ref=out_ref.at[pl.ds(row0, HALF), :],
            dst_ref=out_ref.at[pl.ds(row0, HALF), :],
            send_sem=sems.at[2],
            recv_sem=sems.at[3],
            device_id=y_nbr,
            device_id_type=pl.DeviceIdType.MESH,
        )
        y_rdma.start()
        y_rdma.wait()

    return pl.pallas_call(
        body,
        out_shape=jax.ShapeDtypeStruct((M, D), jnp.float32),
        in_specs=[
            pl.BlockSpec(memory_space=pltpu.VMEM),
            pl.BlockSpec(memory_space=pltpu.VMEM),
            pl.BlockSpec(memory_space=pltpu.VMEM),
        ],
        out_specs=pl.BlockSpec(memory_space=pltpu.VMEM),
        scratch_shapes=[
            pltpu.VMEM((HALF, D), jnp.float32),
            pltpu.SemaphoreType.DMA((4,)),
        ],
        compiler_params=pltpu.CompilerParams(collective_id=0),
    )(partial, resid, gamma2d)


# baseline (device time: 137228 ns/iter reference)
import jax
import jax.numpy as jnp
from jax import lax
from jax.experimental import pallas as pl
from jax.experimental.pallas import tpu as pltpu

M = 2048
D = 2048
HALF = M // 2
C = 4
R = HALF // C


def kernel(partial, resid, gamma):
    gamma2d = gamma.reshape(1, D)

    def body(p_ref, r_ref, g_ref, out_ref,
             xrecv, pin, rin, oc,
             x_send_sems, x_recv_sems, y_send_sems, y_recv_sems,
             pload_sems, rload_sems, ostore_sems):
        my_x = lax.axis_index("x")
        my_y = lax.axis_index("y")
        x_nbr = (1 - my_x, my_y)
        y_nbr = (my_x, 1 - my_y)

        barrier = pltpu.get_barrier_semaphore()
        for nbr in (x_nbr, y_nbr):
            pl.semaphore_signal(
                barrier, inc=1, device_id=nbr,
                device_id_type=pl.DeviceIdType.MESH,
            )
        pl.semaphore_wait(barrier, 2)

        row0 = my_y * HALF

        x_rdmas = []
        for c in range(C):
            rd = pltpu.make_async_remote_copy(
                src_ref=p_ref.at[0, pl.ds(row0 + c * R, R), :],
                dst_ref=xrecv.at[pl.ds(c * R, R), :],
                send_sem=x_send_sems.at[c],
                recv_sem=x_recv_sems.at[c],
                device_id=x_nbr,
                device_id_type=pl.DeviceIdType.MESH,
            )
            rd.start()
            x_rdmas.append(rd)

        y_rdmas = []
        for c in range(C):
            rows = pl.ds(row0 + c * R, R)
            pload = pltpu.make_async_copy(
                p_ref.at[0, rows, :], pin, pload_sems.at[0])
            rload = pltpu.make_async_copy(
                r_ref.at[rows, :], rin, rload_sems.at[0])
            pload.start()
            rload.start()
            pload.wait()
            rload.wait()
            x_rdmas[c].wait_recv()

            y = pin[...] + xrecv[pl.ds(c * R, R), :] + rin[...]
            ms = jnp.mean(y * y, axis=-1, keepdims=True)
            oc[...] = (y * lax.rsqrt(ms + 1e-6)) * g_ref[...]

            ostore = pltpu.make_async_copy(
                oc, out_ref.at[rows, :], ostore_sems.at[0])
            ostore.start()
            yrd = pltpu.make_async_remote_copy(
                src_ref=oc,
                dst_ref=out_ref.at[rows, :],
                send_sem=y_send_sems.at[c],
                recv_sem=y_recv_sems.at[c],
                device_id=y_nbr,
                device_id_type=pl.DeviceIdType.MESH,
            )
            yrd.start()
            ostore.wait()
            yrd.wait_send()
            y_rdmas.append(yrd)

        for c in range(C):
            y_rdmas[c].wait_recv()
            x_rdmas[c].wait_send()

    return pl.pallas_call(
        body,
        out_shape=jax.ShapeDtypeStruct((M, D), jnp.float32),
        in_specs=[
            pl.BlockSpec(memory_space=pltpu.MemorySpace.HBM),
            pl.BlockSpec(memory_space=pltpu.MemorySpace.HBM),
            pl.BlockSpec(memory_space=pltpu.VMEM),
        ],
        out_specs=pl.BlockSpec(memory_space=pltpu.MemorySpace.HBM),
        scratch_shapes=[
            pltpu.VMEM((HALF, D), jnp.float32),
            pltpu.VMEM((R, D), jnp.float32),
            pltpu.VMEM((R, D), jnp.float32),
            pltpu.VMEM((R, D), jnp.float32),
            pltpu.SemaphoreType.DMA((C,)),
            pltpu.SemaphoreType.DMA((C,)),
            pltpu.SemaphoreType.DMA((C,)),
            pltpu.SemaphoreType.DMA((C,)),
            pltpu.SemaphoreType.DMA((1,)),
            pltpu.SemaphoreType.DMA((1,)),
            pltpu.SemaphoreType.DMA((1,)),
        ],
        compiler_params=pltpu.CompilerParams(collective_id=0),
    )(partial, resid, gamma2d)


# device time: 109826 ns/iter; 1.2495x vs baseline; 1.2495x over previous
import jax
import jax.numpy as jnp
from jax import lax
from jax.experimental import pallas as pl
from jax.experimental.pallas import tpu as pltpu

M = 2048
D = 2048
HALF = M // 2
C = 16
R = HALF // C


def kernel(partial, resid, gamma):
    gamma2d = gamma.reshape(1, D)

    def body(p_ref, r_ref, g_ref, out_ref,
             xrecv, pin, rin, oc,
             x_send_sems, x_recv_sems, y_send_sems, y_recv_sems,
             pload_sems, rload_sems, ostore_sems):
        my_x = lax.axis_index("x")
        my_y = lax.axis_index("y")
        x_nbr = (1 - my_x, my_y)
        y_nbr = (my_x, 1 - my_y)

        barrier = pltpu.get_barrier_semaphore()
        for nbr in (x_nbr, y_nbr):
            pl.semaphore_signal(
                barrier, inc=1, device_id=nbr,
                device_id_type=pl.DeviceIdType.MESH,
            )
        pl.semaphore_wait(barrier, 2)

        row0 = my_y * HALF

        x_rdmas = []
        for c in range(C):
            rd = pltpu.make_async_remote_copy(
                src_ref=p_ref.at[0, pl.ds(row0 + c * R, R), :],
                dst_ref=xrecv.at[pl.ds(c * R, R), :],
                send_sem=x_send_sems.at[c],
                recv_sem=x_recv_sems.at[c],
                device_id=x_nbr,
                device_id_type=pl.DeviceIdType.MESH,
            )
            rd.start()
            x_rdmas.append(rd)

        def make_loads(c):
            slot = c % 2
            rows = pl.ds(row0 + c * R, R)
            pload = pltpu.make_async_copy(
                p_ref.at[0, rows, :], pin.at[slot], pload_sems.at[slot])
            rload = pltpu.make_async_copy(
                r_ref.at[rows, :], rin.at[slot], rload_sems.at[slot])
            pload.start()
            rload.start()
            return pload, rload

        loads = {0: make_loads(0)}
        y_rdmas = []
        ostores = []
        for c in range(C):
            slot = c % 2
            rows = pl.ds(row0 + c * R, R)
            if c + 1 < C:
                loads[c + 1] = make_loads(c + 1)
            pload, rload = loads.pop(c)
            pload.wait()
            rload.wait()
            x_rdmas[c].wait_recv()

            if c >= 2:
                ostores[c - 2].wait()
                y_rdmas[c - 2].wait_send()

            y = pin[slot] + xrecv[pl.ds(c * R, R), :] + rin[slot]
            ms = jnp.mean(y * y, axis=-1, keepdims=True)
            oc[slot] = (y * lax.rsqrt(ms + 1e-6)) * g_ref[...]

            ostore = pltpu.make_async_copy(
                oc.at[slot], out_ref.at[rows, :], ostore_sems.at[slot])
            ostore.start()
            ostores.append(ostore)
            yrd = pltpu.make_async_remote_copy(
                src_ref=oc.at[slot],
                dst_ref=out_ref.at[rows, :],
                send_sem=y_send_sems.at[c],
                recv_sem=y_recv_sems.at[c],
                device_id=y_nbr,
                device_id_type=pl.DeviceIdType.MESH,
            )
            yrd.start()
            y_rdmas.append(yrd)

        for c in (C - 2, C - 1):
            ostores[c].wait()
            y_rdmas[c].wait_send()
        for c in range(C):
            y_rdmas[c].wait_recv()
            x_rdmas[c].wait_send()

    return pl.pallas_call(
        body,
        out_shape=jax.ShapeDtypeStruct((M, D), jnp.float32),
        in_specs=[
            pl.BlockSpec(memory_space=pltpu.MemorySpace.HBM),
            pl.BlockSpec(memory_space=pltpu.MemorySpace.HBM),
            pl.BlockSpec(memory_space=pltpu.VMEM),
        ],
        out_specs=pl.BlockSpec(memory_space=pltpu.MemorySpace.HBM),
        scratch_shapes=[
            pltpu.VMEM((HALF, D), jnp.float32),
            pltpu.VMEM((2, R, D), jnp.float32),
            pltpu.VMEM((2, R, D), jnp.float32),
            pltpu.VMEM((2, R, D), jnp.float32),
            pltpu.SemaphoreType.DMA((C,)),
            pltpu.SemaphoreType.DMA((C,)),
            pltpu.SemaphoreType.DMA((C,)),
            pltpu.SemaphoreType.DMA((C,)),
            pltpu.SemaphoreType.DMA((2,)),
            pltpu.SemaphoreType.DMA((2,)),
            pltpu.SemaphoreType.DMA((2,)),
        ],
        compiler_params=pltpu.CompilerParams(collective_id=0),
    )(partial, resid, gamma2d)


# device time: 106940 ns/iter; 1.2832x vs baseline; 1.0270x over previous
import jax
import jax.numpy as jnp
from jax import lax
from jax.experimental import pallas as pl
from jax.experimental.pallas import tpu as pltpu

M = 2048
D = 2048
HALF = M // 2
C = 32
R = HALF // C


def kernel(partial, resid, gamma):
    gamma2d = gamma.reshape(1, D)

    def body(p_ref, r_ref, g_ref, out_ref,
             xrecv, pin, rin, oc,
             x_send_sems, x_recv_sems, y_send_sems, y_recv_sems,
             pload_sems, rload_sems, ostore_sems):
        my_x = lax.axis_index("x")
        my_y = lax.axis_index("y")
        x_nbr = (1 - my_x, my_y)
        y_nbr = (my_x, 1 - my_y)

        barrier = pltpu.get_barrier_semaphore()
        for nbr in (x_nbr, y_nbr):
            pl.semaphore_signal(
                barrier, inc=1, device_id=nbr,
                device_id_type=pl.DeviceIdType.MESH,
            )
        pl.semaphore_wait(barrier, 2)

        row0 = my_y * HALF

        x_rdmas = []
        for c in range(C):
            rd = pltpu.make_async_remote_copy(
                src_ref=p_ref.at[0, pl.ds(row0 + c * R, R), :],
                dst_ref=xrecv.at[pl.ds(c * R, R), :],
                send_sem=x_send_sems.at[c],
                recv_sem=x_recv_sems.at[c],
                device_id=x_nbr,
                device_id_type=pl.DeviceIdType.MESH,
            )
            rd.start()
            x_rdmas.append(rd)

        def make_loads(c):
            slot = c % 2
            rows = pl.ds(row0 + c * R, R)
            pload = pltpu.make_async_copy(
                p_ref.at[0, rows, :], pin.at[slot], pload_sems.at[slot])
            rload = pltpu.make_async_copy(
                r_ref.at[rows, :], rin.at[slot], rload_sems.at[slot])
            pload.start()
            rload.start()
            return pload, rload

        loads = {0: make_loads(0)}
        y_rdmas = []
        ostores = []
        for c in range(C):
            slot = c % 2
            rows = pl.ds(row0 + c * R, R)
            if c + 1 < C:
                loads[c + 1] = make_loads(c + 1)
            pload, rload = loads.pop(c)
            pload.wait()
            rload.wait()
            x_rdmas[c].wait_recv()

            if c >= 2:
                ostores[c - 2].wait()
                y_rdmas[c - 2].wait_send()

            y = pin[slot] + xrecv[pl.ds(c * R, R), :] + rin[slot]
            ms = jnp.mean(y * y, axis=-1, keepdims=True)
            oc[slot] = (y * lax.rsqrt(ms + 1e-6)) * g_ref[...]

            ostore = pltpu.make_async_copy(
                oc.at[slot], out_ref.at[rows, :], ostore_sems.at[slot])
            ostore.start()
            ostores.append(ostore)
            yrd = pltpu.make_async_remote_copy(
                src_ref=oc.at[slot],
                dst_ref=out_ref.at[rows, :],
                send_sem=y_send_sems.at[c],
                recv_sem=y_recv_sems.at[c],
                device_id=y_nbr,
                device_id_type=pl.DeviceIdType.MESH,
            )
            yrd.start()
            y_rdmas.append(yrd)

        for c in (C - 2, C - 1):
            ostores[c].wait()
            y_rdmas[c].wait_send()
        for c in range(C):
            y_rdmas[c].wait_recv()
            x_rdmas[c].wait_send()

    return pl.pallas_call(
        body,
        out_shape=jax.ShapeDtypeStruct((M, D), jnp.float32),
        in_specs=[
            pl.BlockSpec(memory_space=pltpu.MemorySpace.HBM),
            pl.BlockSpec(memory_space=pltpu.MemorySpace.HBM),
            pl.BlockSpec(memory_space=pltpu.VMEM),
        ],
        out_specs=pl.BlockSpec(memory_space=pltpu.MemorySpace.HBM),
        scratch_shapes=[
            pltpu.VMEM((HALF, D), jnp.float32),
            pltpu.VMEM((2, R, D), jnp.float32),
            pltpu.VMEM((2, R, D), jnp.float32),
            pltpu.VMEM((2, R, D), jnp.float32),
            pltpu.SemaphoreType.DMA((C,)),
            pltpu.SemaphoreType.DMA((C,)),
            pltpu.SemaphoreType.DMA((C,)),
            pltpu.SemaphoreType.DMA((C,)),
            pltpu.SemaphoreType.DMA((2,)),
            pltpu.SemaphoreType.DMA((2,)),
            pltpu.SemaphoreType.DMA((2,)),
        ],
        compiler_params=pltpu.CompilerParams(collective_id=0),
    )(partial, resid, gamma2d)


# device time: 106774 ns/iter; 1.2852x vs baseline; 1.0016x over previous
import jax
import jax.numpy as jnp
from jax import lax
from jax.experimental import pallas as pl
from jax.experimental.pallas import tpu as pltpu

M = 2048
D = 2048
HALF = M // 2
C = 32
R = HALF // C


def kernel(partial, resid, gamma):
    gamma2d = gamma.reshape(1, D)

    def body(p_ref, r_ref, g_ref, out_ref,
             xrecv, pin, rin,
             x_send_sems, x_recv_sems, y_send_sems, y_recv_sems,
             pload_sems, rload_sems):
        my_x = lax.axis_index("x")
        my_y = lax.axis_index("y")
        x_nbr = (1 - my_x, my_y)
        y_nbr = (my_x, 1 - my_y)

        row0 = my_y * HALF

        def make_loads(c):
            slot = c % 2
            rows = pl.ds(row0 + c * R, R)
            pload = pltpu.make_async_copy(
                p_ref.at[0, rows, :], pin.at[slot], pload_sems.at[slot])
            rload = pltpu.make_async_copy(
                r_ref.at[rows, :], rin.at[slot], rload_sems.at[slot])
            pload.start()
            rload.start()
            return pload, rload

        loads = {0: make_loads(0)}

        barrier = pltpu.get_barrier_semaphore()
        for nbr in (x_nbr, y_nbr):
            pl.semaphore_signal(
                barrier, inc=1, device_id=nbr,
                device_id_type=pl.DeviceIdType.MESH,
            )
        pl.semaphore_wait(barrier, 2)

        x_rdmas = []
        for c in range(C):
            rd = pltpu.make_async_remote_copy(
                src_ref=p_ref.at[0, pl.ds(row0 + c * R, R), :],
                dst_ref=xrecv.at[pl.ds(c * R, R), :],
                send_sem=x_send_sems.at[c],
                recv_sem=x_recv_sems.at[c],
                device_id=x_nbr,
                device_id_type=pl.DeviceIdType.MESH,
            )
            rd.start()
            x_rdmas.append(rd)

        y_rdmas = []
        for c in range(C):
            slot = c % 2
            rows = pl.ds(row0 + c * R, R)
            if c + 1 < C:
                loads[c + 1] = make_loads(c + 1)
            pload, rload = loads.pop(c)
            pload.wait()
            rload.wait()
            x_rdmas[c].wait_recv()

            y = pin[slot] + xrecv[pl.ds(c * R, R), :] + rin[slot]
            ms = jnp.mean(y * y, axis=-1, keepdims=True)
            out_ref[rows, :] = (y * lax.rsqrt(ms + 1e-6)) * g_ref[...]

            yrd = pltpu.make_async_remote_copy(
                src_ref=out_ref.at[rows, :],
                dst_ref=out_ref.at[rows, :],
                send_sem=y_send_sems.at[c],
                recv_sem=y_recv_sems.at[c],
                device_id=y_nbr,
                device_id_type=pl.DeviceIdType.MESH,
            )
            yrd.start()
            y_rdmas.append(yrd)

        for c in range(C):
            y_rdmas[c].wait_recv()
            y_rdmas[c].wait_send()
            x_rdmas[c].wait_send()

    return pl.pallas_call(
        body,
        out_shape=jax.ShapeDtypeStruct((M, D), jnp.float32),
        in_specs=[
            pl.BlockSpec(memory_space=pltpu.MemorySpace.HBM),
            pl.BlockSpec(memory_space=pltpu.MemorySpace.HBM),
            pl.BlockSpec(memory_space=pltpu.VMEM),
        ],
        out_specs=pl.BlockSpec(memory_space=pltpu.VMEM),
        scratch_shapes=[
            pltpu.VMEM((HALF, D), jnp.float32),
            pltpu.VMEM((2, R, D), jnp.float32),
            pltpu.VMEM((2, R, D), jnp.float32),
            pltpu.SemaphoreType.DMA((C,)),
            pltpu.SemaphoreType.DMA((C,)),
            pltpu.SemaphoreType.DMA((C,)),
            pltpu.SemaphoreType.DMA((C,)),
            pltpu.SemaphoreType.DMA((2,)),
            pltpu.SemaphoreType.DMA((2,)),
        ],
        compiler_params=pltpu.CompilerParams(collective_id=0),
    )(partial, resid, gamma2d)
